# baseline (speedup 1.0000x reference)
import functools

import jax
import jax.numpy as jnp
from jax import lax
from jax.experimental import pallas as pl
from jax.experimental.pallas import tpu as pltpu
from jax.experimental.pallas import tpu_sc as plsc

N = 10000
E = 160000
D = 256
H = 128
LOOPS = 2
THR1 = 0.5
EPS = 1e-5

NT = 16
EPT = E // NT
K = 40
NCHUNK = EPT // K
NB = 10
BR = N // NB

_f32 = jnp.float32


def _prep_body(wt_ref, bt_ref, wg_ref, bg_ref, wd_ref, bd_ref, ws_ref):
    wp = wt_ref[:D, :] - wt_ref[D:, :]
    wq = wt_ref[D:, :]
    wg1 = wg_ref[:D, :] - wg_ref[D:, :]
    wg2 = wg_ref[D:, :]
    for c in range(2):
        cs = slice(c * H, (c + 1) * H)
        wd_ref[:, 2 * c * H:(2 * c + 1) * H] = wp[:, cs]
        wd_ref[:, (2 * c + 1) * H:(2 * c + 2) * H] = wg1[:, cs]
        ws_ref[:, 2 * c * H:(2 * c + 1) * H] = wq[:, cs]
        ws_ref[:, (2 * c + 1) * H:(2 * c + 2) * H] = wg2[:, cs]
        bd_ref[:, 2 * c * H:(2 * c + 1) * H] = bt_ref[:, cs]
        bd_ref[:, (2 * c + 1) * H:(2 * c + 2) * H] = bg_ref[:, cs]


_prep = pl.pallas_call(
    _prep_body,
    out_shape=[
        jax.ShapeDtypeStruct((D, 2 * D), _f32),
        jax.ShapeDtypeStruct((1, 2 * D), _f32),
        jax.ShapeDtypeStruct((D, 2 * D), _f32),
    ],
)


def _tables_body(h_ref, wd_ref, bd_ref, ws_ref, dt0_ref, dt1_ref,
                 st0_ref, st1_ref):
    h = h_ref[...]
    hw_d = jnp.dot(h, wd_ref[...], preferred_element_type=_f32) + bd_ref[...]
    hw_s = jnp.dot(h, ws_ref[...], preferred_element_type=_f32)
    dt0_ref[...] = hw_d[:, :D]
    dt1_ref[...] = hw_d[:, D:]
    st0_ref[...] = hw_s[:, :D]
    st1_ref[...] = hw_s[:, D:]


_tables = pl.pallas_call(
    _tables_body,
    grid=(NB,),
    in_specs=[
        pl.BlockSpec((BR, D), lambda j: (j, 0)),
        pl.BlockSpec((D, 2 * D), lambda j: (0, 0)),
        pl.BlockSpec((1, 2 * D), lambda j: (0, 0)),
        pl.BlockSpec((D, 2 * D), lambda j: (0, 0)),
    ],
    out_specs=[pl.BlockSpec((BR, D), lambda j: (j, 0))] * 4,
    out_shape=[jax.ShapeDtypeStruct((N, D), _f32)] * 4,
)


def _edge_body(dtab0, dtab1, stab0, stab1, dst_e, src_e, out,
               dst_v, src_v, drows, srows, contrib, agg_sh, sem_d, sem_s):
    c = lax.axis_index("c")
    s = lax.axis_index("s")
    zeros16 = jnp.zeros((16,), _f32)

    nown = jnp.where(s < 10, 16, 15)
    chunk0 = 16 * s - jnp.maximum(s - 10, 0)

    @pl.loop(0, K)
    def _zero_rows(r):
        for i in range(H // 16):
            contrib[r, pl.ds(i * 16, 16)] = zeros16

    def _zbody(r, _):
        pltpu.sync_copy(contrib, agg_sh.at[pl.ds((chunk0 + r) * K, K)])
        return 0

    lax.fori_loop(0, nown, _zbody, 0)
    plsc.subcore_barrier()

    def _process(dtab, stab):
        @pl.loop(0, NCHUNK)
        def _chunk(k):
            off = s * EPT + k * K
            pltpu.sync_copy(dst_e.at[pl.ds(off, K)], dst_v)
            pltpu.sync_copy(src_e.at[pl.ds(off, K)], src_v)
            cp_d = pltpu.async_copy(dtab.at[dst_v], drows, sem_d)
            cp_s = pltpu.async_copy(stab.at[src_v], srows, sem_s)
            cp_d.wait()
            cp_s.wait()

            @pl.loop(0, K)
            def _row(r):
                for i in range(H // 16):
                    sl = pl.ds(i * 16, 16)
                    sg = pl.ds(H + i * 16, 16)
                    m = drows[r, sl] + srows[r, sl]
                    gl = drows[r, sg] + srows[r, sg]
                    g = 1.0 / (1.0 + jnp.exp(-gl))
                    contrib[r, sl] = g * m

            pltpu.sync_copy(contrib, agg_sh.at[dst_v], add=True)

    @pl.when(c == 0)
    def _half0():
        _process(dtab0, stab0)

    @pl.when(c == 1)
    def _half1():
        _process(dtab1, stab1)

    plsc.subcore_barrier()

    def _cbody(r, _):
        row0 = (chunk0 + r) * K
        pltpu.sync_copy(agg_sh.at[pl.ds(row0, K)], contrib)
        pltpu.sync_copy(contrib, out.at[c, pl.ds(row0, K)])
        return 0

    lax.fori_loop(0, nown, _cbody, 0)


_edges = pl.kernel(
    _edge_body,
    out_type=jax.ShapeDtypeStruct((2, N, H), _f32),
    mesh=plsc.VectorSubcoreMesh(core_axis_name="c", subcore_axis_name="s"),
    scratch_types=[
        pltpu.VMEM((K,), jnp.int32),
        pltpu.VMEM((K,), jnp.int32),
        pltpu.VMEM((K, D), _f32),
        pltpu.VMEM((K, D), _f32),
        pltpu.VMEM((K, H), _f32),
        pltpu.VMEM_SHARED((N, H), _f32),
        pltpu.SemaphoreType.DMA,
        pltpu.SemaphoreType.DMA,
    ],
)


def _bn_stats_body(agg_ref, out_ref, acc_ref):
    j = pl.program_id(0)
    a = jnp.concatenate([agg_ref[0], agg_ref[1]], axis=1)

    @pl.when(j == 0)
    def _():
        acc_ref[...] = jnp.zeros_like(acc_ref)

    acc_ref[0:1, :] += jnp.sum(a, axis=0, keepdims=True)
    acc_ref[1:2, :] += jnp.sum(a * a, axis=0, keepdims=True)
    out_ref[...] = acc_ref[...]


_bn_stats = pl.pallas_call(
    _bn_stats_body,
    grid=(NB,),
    in_specs=[pl.BlockSpec((2, BR, H), lambda j: (0, j, 0))],
    out_specs=pl.BlockSpec((2, D), lambda j: (0, 0)),
    out_shape=jax.ShapeDtypeStruct((2, D), _f32),
    scratch_shapes=[pltpu.VMEM((2, D), _f32)],
)


def _bn_core(agg_ref, h_ref, st_ref, gam_ref, bet_ref):
    a = jnp.concatenate([agg_ref[0], agg_ref[1]], axis=1)
    mu = st_ref[0:1, :] * (1.0 / N)
    var = st_ref[1:2, :] * (1.0 / N) - mu * mu
    inv = lax.rsqrt(var + EPS)
    hn = gam_ref[...] * ((a - mu) * inv) + bet_ref[...]
    return jnp.maximum(hn, 0.0) + h_ref[...]


def _bn_apply_body(agg_ref, h_ref, st_ref, gam_ref, bet_ref, out_ref):
    out_ref[...] = _bn_core(agg_ref, h_ref, st_ref, gam_ref, bet_ref)


def _bn_apply_sum_body(agg_ref, h_ref, st_ref, gam_ref, bet_ref,
                       out_ref, hsum_ref, acc_ref):
    j = pl.program_id(0)
    hnew = _bn_core(agg_ref, h_ref, st_ref, gam_ref, bet_ref)
    out_ref[...] = hnew

    @pl.when(j == 0)
    def _():
        acc_ref[...] = jnp.zeros_like(acc_ref)

    acc_ref[...] += jnp.sum(hnew, axis=0, keepdims=True)
    hsum_ref[...] = acc_ref[...]


_bn_in_specs = [
    pl.BlockSpec((2, BR, H), lambda j: (0, j, 0)),
    pl.BlockSpec((BR, D), lambda j: (j, 0)),
    pl.BlockSpec((2, D), lambda j: (0, 0)),
    pl.BlockSpec((1, D), lambda j: (0, 0)),
    pl.BlockSpec((1, D), lambda j: (0, 0)),
]

_bn_apply = pl.pallas_call(
    _bn_apply_body,
    grid=(NB,),
    in_specs=_bn_in_specs,
    out_specs=pl.BlockSpec((BR, D), lambda j: (j, 0)),
    out_shape=jax.ShapeDtypeStruct((N, D), _f32),
)

_bn_apply_sum = pl.pallas_call(
    _bn_apply_sum_body,
    grid=(NB,),
    in_specs=_bn_in_specs,
    out_specs=[
        pl.BlockSpec((BR, D), lambda j: (j, 0)),
        pl.BlockSpec((1, D), lambda j: (0, 0)),
    ],
    out_shape=[
        jax.ShapeDtypeStruct((N, D), _f32),
        jax.ShapeDtypeStruct((1, D), _f32),
    ],
    scratch_shapes=[pltpu.VMEM((1, D), _f32)],
)


def _ca_body(hsum_ref, w1_ref, b1_ref, w2_ref, b2_ref, out_ref):
    sm = hsum_ref[...] * (1.0 / N)
    t = jnp.dot(sm, w1_ref[...], preferred_element_type=_f32) + b1_ref[...]
    t = jnp.maximum(t, 0.0)
    u = jnp.dot(t, w2_ref[...], preferred_element_type=_f32) + b2_ref[...]
    ca = 1.0 / (1.0 + jnp.exp(-u))
    out_ref[...] = jnp.where(ca > THR1, ca, 0.0)


_ca = pl.pallas_call(
    _ca_body,
    out_shape=jax.ShapeDtypeStruct((1, D), _f32),
)


def _out_body(h_ref, ca_ref, wf_ref, bf_ref, out_ref):
    hca = h_ref[...] * ca_ref[...]
    out_ref[...] = (
        jnp.dot(hca, wf_ref[...], preferred_element_type=_f32) + bf_ref[...]
    )


_out = pl.pallas_call(
    _out_body,
    grid=(NB,),
    in_specs=[
        pl.BlockSpec((BR, D), lambda j: (j, 0)),
        pl.BlockSpec((1, D), lambda j: (0, 0)),
        pl.BlockSpec((D, D), lambda j: (0, 0)),
        pl.BlockSpec((1, D), lambda j: (0, 0)),
    ],
    out_specs=pl.BlockSpec((BR, D), lambda j: (j, 0)),
    out_shape=jax.ShapeDtypeStruct((N, D), _f32),
)


def kernel(x, edge_index, W_theta, b_theta, W_gate, b_gate, gamma, beta,
           W_ca1, b_ca1, W_ca2, b_ca2, W_fu, b_fu):
    bt = b_theta.reshape(1, D)
    bg = b_gate.reshape(1, D)
    gam = gamma.reshape(1, D)
    bet = beta.reshape(1, D)

    wd, bd, ws = _prep(W_theta, bt, W_gate, bg)

    h = x
    hsum = None
    for i in range(LOOPS):
        dt0, dt1, st0, st1 = _tables(h, wd, bd, ws)
        agg = _edges(dt0, dt1, st0, st1, edge_index[1], edge_index[0])
        stats = _bn_stats(agg)
        if i < LOOPS - 1:
            h = _bn_apply(agg, h, stats, gam, bet)
        else:
            h, hsum = _bn_apply_sum(agg, h, stats, gam, bet)

    ca = _ca(hsum, W_ca1, b_ca1.reshape(1, D), W_ca2, b_ca2.reshape(1, D))
    return _out(h, ca, W_fu, b_fu.reshape(1, D))


# speedup vs baseline: 1.2543x; 1.2543x over previous
import functools

import jax
import jax.numpy as jnp
from jax import lax
from jax.experimental import pallas as pl
from jax.experimental.pallas import tpu as pltpu
from jax.experimental.pallas import tpu_sc as plsc

N = 10000
E = 160000
D = 256
H = 128
LOOPS = 2
THR1 = 0.5
EPS = 1e-5

NT = 16
EPT = E // NT
K = 40
NCHUNK = EPT // K
NBLK = 25
BE = EPT // NBLK
CPB = BE // K
NB = 10
BR = N // NB

_f32 = jnp.float32


def _prep_body(wt_ref, bt_ref, wg_ref, bg_ref, wd_ref, bd_ref, ws_ref):
    wp = wt_ref[:D, :] - wt_ref[D:, :]
    wq = wt_ref[D:, :]
    wg1 = wg_ref[:D, :] - wg_ref[D:, :]
    wg2 = wg_ref[D:, :]
    for c in range(2):
        cs = slice(c * H, (c + 1) * H)
        wd_ref[:, 2 * c * H:(2 * c + 1) * H] = wp[:, cs]
        wd_ref[:, (2 * c + 1) * H:(2 * c + 2) * H] = wg1[:, cs]
        ws_ref[:, 2 * c * H:(2 * c + 1) * H] = wq[:, cs]
        ws_ref[:, (2 * c + 1) * H:(2 * c + 2) * H] = wg2[:, cs]
        bd_ref[:, 2 * c * H:(2 * c + 1) * H] = bt_ref[:, cs]
        bd_ref[:, (2 * c + 1) * H:(2 * c + 2) * H] = bg_ref[:, cs]


_prep = pl.pallas_call(
    _prep_body,
    out_shape=[
        jax.ShapeDtypeStruct((D, 2 * D), _f32),
        jax.ShapeDtypeStruct((1, 2 * D), _f32),
        jax.ShapeDtypeStruct((D, 2 * D), _f32),
    ],
)


def _tables_body(h_ref, wd_ref, bd_ref, ws_ref, dt0_ref, dt1_ref,
                 st0_ref, st1_ref):
    h = h_ref[...]
    hw_d = jnp.dot(h, wd_ref[...], preferred_element_type=_f32) + bd_ref[...]
    hw_s = jnp.dot(h, ws_ref[...], preferred_element_type=_f32)
    dt0_ref[...] = hw_d[:, :D]
    dt1_ref[...] = hw_d[:, D:]
    st0_ref[...] = hw_s[:, :D]
    st1_ref[...] = hw_s[:, D:]


_tables = pl.pallas_call(
    _tables_body,
    grid=(NB,),
    in_specs=[
        pl.BlockSpec((BR, D), lambda j: (j, 0)),
        pl.BlockSpec((D, 2 * D), lambda j: (0, 0)),
        pl.BlockSpec((1, 2 * D), lambda j: (0, 0)),
        pl.BlockSpec((D, 2 * D), lambda j: (0, 0)),
    ],
    out_specs=[pl.BlockSpec((BR, D), lambda j: (j, 0))] * 4,
    out_shape=[jax.ShapeDtypeStruct((N, D), _f32)] * 4,
)


def _edge_body(dtab0, dtab1, stab0, stab1, dst_e, src_e, out,
               dstblk, srcblk, drows0, drows1, srows0, srows1,
               contrib, agg_sh,
               sem_gd0, sem_gd1, sem_gs0, sem_gs1, sem_sc):
    c = lax.axis_index("c")
    s = lax.axis_index("s")
    zeros16 = jnp.zeros((16,), _f32)
    rowsd = (drows0, drows1)
    rowss = (srows0, srows1)
    sem_gd = (sem_gd0, sem_gd1)
    sem_gs = (sem_gs0, sem_gs1)

    nown = jnp.where(s < 10, 16, 15)
    chunk0 = 16 * s - jnp.maximum(s - 10, 0)

    @pl.loop(0, K)
    def _zero_rows(r):
        for i in range(H // 16):
            contrib[r, pl.ds(i * 16, 16)] = zeros16

    def _zbody(r, _):
        pltpu.sync_copy(contrib, agg_sh.at[pl.ds((chunk0 + r) * K, K)])
        return 0

    lax.fori_loop(0, nown, _zbody, 0)
    plsc.subcore_barrier()

    def _process(dtab, stab):
        def issue_gather(j, p):
            pltpu.async_copy(dtab.at[dstblk.at[j]], rowsd[p], sem_gd[p])
            pltpu.async_copy(stab.at[srcblk.at[j]], rowss[p], sem_gs[p])

        def wait_gather(j, p):
            pltpu.make_async_copy(dtab.at[dstblk.at[j]], rowsd[p],
                                  sem_gd[p]).wait()
            pltpu.make_async_copy(stab.at[srcblk.at[j]], rowss[p],
                                  sem_gs[p]).wait()

        def wait_scatter(j):
            pltpu.make_async_copy(contrib, agg_sh.at[dstblk.at[j]],
                                  sem_sc).wait()

        @pl.loop(0, NBLK)
        def _blk(b):
            @pl.when(b > 0)
            def _drain_prev_block():
                wait_scatter(CPB - 1)

            pltpu.sync_copy(dst_e.at[s, b], dstblk)
            pltpu.sync_copy(src_e.at[s, b], srcblk)
            issue_gather(0, 0)

            @pl.loop(0, CPB // 2)
            def _jo(jo):
                for p in range(2):
                    j = jo * 2 + p
                    wait_gather(j, p)

                    @pl.when(j < CPB - 1)
                    def _prefetch():
                        issue_gather(j + 1, 1 - p)

                    if p == 0:
                        @pl.when(jo > 0)
                        def _drain0():
                            wait_scatter(j - 1)
                    else:
                        wait_scatter(j - 1)

                    @pl.loop(0, K, unroll=2)
                    def _row(r):
                        for i in range(H // 16):
                            sl = pl.ds(i * 16, 16)
                            sg = pl.ds(H + i * 16, 16)
                            m = rowsd[p][r, sl] + rowss[p][r, sl]
                            gl = rowsd[p][r, sg] + rowss[p][r, sg]
                            g = 1.0 / (1.0 + jnp.exp(-gl))
                            contrib[r, sl] = g * m

                    pltpu.async_copy(contrib, agg_sh.at[dstblk.at[j]],
                                     sem_sc, add=True)

        wait_scatter(CPB - 1)

    @pl.when(c == 0)
    def _half0():
        _process(dtab0, stab0)

    @pl.when(c == 1)
    def _half1():
        _process(dtab1, stab1)

    plsc.subcore_barrier()

    def _cbody(r, _):
        row0 = (chunk0 + r) * K
        pltpu.sync_copy(agg_sh.at[pl.ds(row0, K)], contrib)
        pltpu.sync_copy(contrib, out.at[c, pl.ds(row0, K)])
        return 0

    lax.fori_loop(0, nown, _cbody, 0)


_edges = pl.kernel(
    _edge_body,
    out_type=jax.ShapeDtypeStruct((2, N, H), _f32),
    mesh=plsc.VectorSubcoreMesh(core_axis_name="c", subcore_axis_name="s"),
    scratch_types=[
        pltpu.VMEM((CPB, K), jnp.int32),
        pltpu.VMEM((CPB, K), jnp.int32),
        pltpu.VMEM((K, D), _f32),
        pltpu.VMEM((K, D), _f32),
        pltpu.VMEM((K, D), _f32),
        pltpu.VMEM((K, D), _f32),
        pltpu.VMEM((K, H), _f32),
        pltpu.VMEM_SHARED((N, H), _f32),
        pltpu.SemaphoreType.DMA,
        pltpu.SemaphoreType.DMA,
        pltpu.SemaphoreType.DMA,
        pltpu.SemaphoreType.DMA,
        pltpu.SemaphoreType.DMA,
    ],
)


def _bn_stats_body(agg_ref, out_ref, acc_ref):
    j = pl.program_id(0)
    a = jnp.concatenate([agg_ref[0], agg_ref[1]], axis=1)

    @pl.when(j == 0)
    def _():
        acc_ref[...] = jnp.zeros_like(acc_ref)

    acc_ref[0:1, :] += jnp.sum(a, axis=0, keepdims=True)
    acc_ref[1:2, :] += jnp.sum(a * a, axis=0, keepdims=True)
    out_ref[...] = acc_ref[...]


_bn_stats = pl.pallas_call(
    _bn_stats_body,
    grid=(NB,),
    in_specs=[pl.BlockSpec((2, BR, H), lambda j: (0, j, 0))],
    out_specs=pl.BlockSpec((2, D), lambda j: (0, 0)),
    out_shape=jax.ShapeDtypeStruct((2, D), _f32),
    scratch_shapes=[pltpu.VMEM((2, D), _f32)],
)


def _bn_core(agg_ref, h_ref, st_ref, gam_ref, bet_ref):
    a = jnp.concatenate([agg_ref[0], agg_ref[1]], axis=1)
    mu = st_ref[0:1, :] * (1.0 / N)
    var = st_ref[1:2, :] * (1.0 / N) - mu * mu
    inv = lax.rsqrt(var + EPS)
    hn = gam_ref[...] * ((a - mu) * inv) + bet_ref[...]
    return jnp.maximum(hn, 0.0) + h_ref[...]


def _bn_apply_body(agg_ref, h_ref, st_ref, gam_ref, bet_ref, out_ref):
    out_ref[...] = _bn_core(agg_ref, h_ref, st_ref, gam_ref, bet_ref)


def _bn_apply_sum_body(agg_ref, h_ref, st_ref, gam_ref, bet_ref,
                       out_ref, hsum_ref, acc_ref):
    j = pl.program_id(0)
    hnew = _bn_core(agg_ref, h_ref, st_ref, gam_ref, bet_ref)
    out_ref[...] = hnew

    @pl.when(j == 0)
    def _():
        acc_ref[...] = jnp.zeros_like(acc_ref)

    acc_ref[...] += jnp.sum(hnew, axis=0, keepdims=True)
    hsum_ref[...] = acc_ref[...]


_bn_in_specs = [
    pl.BlockSpec((2, BR, H), lambda j: (0, j, 0)),
    pl.BlockSpec((BR, D), lambda j: (j, 0)),
    pl.BlockSpec((2, D), lambda j: (0, 0)),
    pl.BlockSpec((1, D), lambda j: (0, 0)),
    pl.BlockSpec((1, D), lambda j: (0, 0)),
]

_bn_apply = pl.pallas_call(
    _bn_apply_body,
    grid=(NB,),
    in_specs=_bn_in_specs,
    out_specs=pl.BlockSpec((BR, D), lambda j: (j, 0)),
    out_shape=jax.ShapeDtypeStruct((N, D), _f32),
)

_bn_apply_sum = pl.pallas_call(
    _bn_apply_sum_body,
    grid=(NB,),
    in_specs=_bn_in_specs,
    out_specs=[
        pl.BlockSpec((BR, D), lambda j: (j, 0)),
        pl.BlockSpec((1, D), lambda j: (0, 0)),
    ],
    out_shape=[
        jax.ShapeDtypeStruct((N, D), _f32),
        jax.ShapeDtypeStruct((1, D), _f32),
    ],
    scratch_shapes=[pltpu.VMEM((1, D), _f32)],
)


def _ca_body(hsum_ref, w1_ref, b1_ref, w2_ref, b2_ref, out_ref):
    sm = hsum_ref[...] * (1.0 / N)
    t = jnp.dot(sm, w1_ref[...], preferred_element_type=_f32) + b1_ref[...]
    t = jnp.maximum(t, 0.0)
    u = jnp.dot(t, w2_ref[...], preferred_element_type=_f32) + b2_ref[...]
    ca = 1.0 / (1.0 + jnp.exp(-u))
    out_ref[...] = jnp.where(ca > THR1, ca, 0.0)


_ca = pl.pallas_call(
    _ca_body,
    out_shape=jax.ShapeDtypeStruct((1, D), _f32),
)


def _out_body(h_ref, ca_ref, wf_ref, bf_ref, out_ref):
    hca = h_ref[...] * ca_ref[...]
    out_ref[...] = (
        jnp.dot(hca, wf_ref[...], preferred_element_type=_f32) + bf_ref[...]
    )


_out = pl.pallas_call(
    _out_body,
    grid=(NB,),
    in_specs=[
        pl.BlockSpec((BR, D), lambda j: (j, 0)),
        pl.BlockSpec((1, D), lambda j: (0, 0)),
        pl.BlockSpec((D, D), lambda j: (0, 0)),
        pl.BlockSpec((1, D), lambda j: (0, 0)),
    ],
    out_specs=pl.BlockSpec((BR, D), lambda j: (j, 0)),
    out_shape=jax.ShapeDtypeStruct((N, D), _f32),
)


def kernel(x, edge_index, W_theta, b_theta, W_gate, b_gate, gamma, beta,
           W_ca1, b_ca1, W_ca2, b_ca2, W_fu, b_fu):
    bt = b_theta.reshape(1, D)
    bg = b_gate.reshape(1, D)
    gam = gamma.reshape(1, D)
    bet = beta.reshape(1, D)

    wd, bd, ws = _prep(W_theta, bt, W_gate, bg)

    h = x
    hsum = None
    for i in range(LOOPS):
        dt0, dt1, st0, st1 = _tables(h, wd, bd, ws)
        agg = _edges(dt0, dt1, st0, st1,
                     edge_index[1].reshape(NT, NBLK, CPB, K),
                     edge_index[0].reshape(NT, NBLK, CPB, K))
        stats = _bn_stats(agg)
        if i < LOOPS - 1:
            h = _bn_apply(agg, h, stats, gam, bet)
        else:
            h, hsum = _bn_apply_sum(agg, h, stats, gam, bet)

    ca = _ca(hsum, W_ca1, b_ca1.reshape(1, D), W_ca2, b_ca2.reshape(1, D))
    return _out(h, ca, W_fu, b_fu.reshape(1, D))
